# Initial kernel scaffold; baseline (speedup 1.0000x reference)
import jax
import jax.numpy as jnp
from jax.experimental import pallas as pl


def kernel(x, table):
    raise NotImplementedError("write your pallas kernel here")


# speedup vs baseline: 18.7115x; 18.7115x over previous
import functools

import numpy as np
import jax
import jax.numpy as jnp
from jax import lax
from jax.experimental import pallas as pl
from jax.experimental.pallas import tpu as pltpu
from jax.experimental.pallas import tpu_sc as plsc

RES = 1024.0
TS = 4194304
HMASK = np.uint32(TS - 1)
P2 = np.uint32(2654435761)
P3 = np.uint32(805459861)
BATCH = 1048576
NF = 2
CORNERS = [(0, 0, 0), (0, 0, 1), (0, 1, 0), (0, 1, 1),
           (1, 0, 0), (1, 0, 1), (1, 1, 0), (1, 1, 1)]

NC, NS = 2, 16
NW = NC * NS
NPW = BATCH // NW
C = 1024
G = 128
K = C // G
NCHUNK = NPW // C
VG = G // 16

_mesh = plsc.VectorSubcoreMesh(core_axis_name="c", subcore_axis_name="s")


@functools.partial(
    pl.kernel,
    mesh=_mesh,
    out_type=jax.ShapeDtypeStruct((BATCH * NF,), jnp.float32),
    compiler_params=pltpu.CompilerParams(
        needs_layout_passes=False, use_tc_tiling_on_sc=False),
    scratch_types=[
        pltpu.VMEM((3 * C,), jnp.float32),
        *[pltpu.VMEM((G,), jnp.int32) for _ in range(8 * K)],
        *[pltpu.VMEM((C,), jnp.float32) for _ in range(8)],
        *[pltpu.VMEM((C,), jnp.int32) for _ in range(8)],
        *[pltpu.VMEM((G, 8), jnp.float32) for _ in range(8 * K)],
        pltpu.VMEM((NF * C,), jnp.float32),
        pltpu.SemaphoreType.DMA,
    ],
)
def _ingp(x_hbm, table_hbm, out_hbm, xv, *rest):
    idxv = rest[0:8 * K]
    wv = rest[8 * K:8 * K + 8]
    lowv = rest[8 * K + 8:8 * K + 16]
    rows = rest[8 * K + 16:16 * K + 16]
    outv, sem = rest[16 * K + 16:]
    wid = lax.axis_index("s") * NC + lax.axis_index("c")
    iota = lax.iota(jnp.int32, 16)
    iota3 = iota * np.int32(3)
    iota2 = iota * np.int32(2)
    zero16 = jnp.zeros((16,), jnp.int32)
    one16 = jnp.ones((16,), jnp.int32)

    @pl.loop(np.int32(0), np.int32(NCHUNK), init_carry=wid * np.int32(NPW))
    def chunk_body(ci, base):
        base = pl.multiple_of(base, C)
        pltpu.sync_copy(x_hbm.at[pl.ds(base * np.int32(3), 3 * C)], xv)

        for k in range(K):
            @pl.loop(np.int32(0), np.int32(VG), init_carry=np.int32(0))
            def compute_body(i, lo):
                lo = pl.multiple_of(lo, 16)
                o = lo + np.int32(k * G)
                lane3 = o * np.int32(3) + iota3
                t0 = plsc.load_gather(xv, [lane3]) * RES
                t1 = plsc.load_gather(xv, [lane3 + np.int32(1)]) * RES
                t2 = plsc.load_gather(xv, [lane3 + np.int32(2)]) * RES
                c0 = t0.astype(jnp.int32)
                c1 = t1.astype(jnp.int32)
                c2 = t2.astype(jnp.int32)
                f0 = t0 - c0.astype(jnp.float32)
                f1 = t1 - c1.astype(jnp.float32)
                f2 = t2 - c2.astype(jnp.float32)
                u0 = c0.astype(jnp.uint32)
                u1 = c1.astype(jnp.uint32) * P2
                u2 = c2.astype(jnp.uint32) * P3
                u = ((u0, u0 + np.uint32(1)),
                     (u1, u1 + P2),
                     (u2, u2 + P3))
                g0 = 1.0 - f0
                g1 = 1.0 - f1
                g2 = 1.0 - f2
                wyz = {(0, 0): g1 * g2, (0, 1): g1 * f2,
                       (1, 0): f1 * g2, (1, 1): f1 * f2}
                wx = (g0, f0)
                for j, (a, b, c) in enumerate(CORNERS):
                    h = (u[0][a] ^ u[1][b] ^ u[2][c]) & HMASK
                    idxv[j * K + k][pl.ds(lo, 16)] = (
                        h >> np.uint32(2)).astype(jnp.int32)
                    lowv[j][pl.ds(o, 16)] = (
                        (h & np.uint32(3)) * np.uint32(2)).astype(jnp.int32)
                    wv[j][pl.ds(o, 16)] = wx[a] * wyz[(b, c)]
                return lo + np.int32(16)

        descs = []
        for k in range(K):
            for j in range(8):
                descs.append(pltpu.async_copy(
                    table_hbm.at[idxv[j * K + k]],
                    rows[j * K + k],
                    sem,
                ))
        for d in descs:
            d.wait()

        for k in range(K):
            @pl.loop(np.int32(0), np.int32(VG), init_carry=np.int32(0))
            def combine_body(i, lo):
                lo = pl.multiple_of(lo, 16)
                o = lo + np.int32(k * G)
                lidx = lo + iota
                acc0 = jnp.zeros((16,), jnp.float32)
                acc1 = jnp.zeros((16,), jnp.float32)
                for j in range(8):
                    wj = wv[j][pl.ds(o, 16)]
                    col = lowv[j][pl.ds(o, 16)]
                    r = rows[j * K + k]
                    acc0 = acc0 + wj * plsc.load_gather(r, [lidx, col])
                    acc1 = acc1 + wj * plsc.load_gather(r, [lidx, col + one16])
                cid2 = o * np.int32(2) + iota2
                plsc.store_scatter(outv, [cid2], acc0)
                plsc.store_scatter(outv, [cid2 + np.int32(1)], acc1)
                return lo + np.int32(16)

        pltpu.sync_copy(outv, out_hbm.at[pl.ds(base * np.int32(2), NF * C)])
        return base + np.int32(C)


def kernel(x, table):
    from jax._src.config import enable_x64 as _x64_ctx
    with _x64_ctx(False):
        out = _ingp(x.reshape(-1), table.reshape(TS // 4, 8))
    return out.reshape(BATCH, NF)


# speedup vs baseline: 175.0208x; 9.3537x over previous
import functools

import numpy as np
import jax
import jax.numpy as jnp
from jax import lax
from jax.experimental import pallas as pl
from jax.experimental.pallas import tpu as pltpu
from jax.experimental.pallas import tpu_sc as plsc

RES = 1024.0
TS = 4194304
HMASK = np.uint32(TS - 1)
P2 = np.uint32(2654435761)
P3 = np.uint32(805459861)
BATCH = 1048576
NF = 2
PLANE = TS // 8
CORNERS = [(0, 0, 0), (0, 0, 1), (0, 1, 0), (0, 1, 1),
           (1, 0, 0), (1, 0, 1), (1, 1, 0), (1, 1, 1)]

NC, NS = 2, 16
NW = NC * NS
NPW = BATCH // NW
C = 512
CB = C // 128
NCHUNK = NPW // C
NB = BATCH // 128

_mesh = plsc.VectorSubcoreMesh(core_axis_name="c", subcore_axis_name="s")


@functools.partial(
    pl.kernel,
    mesh=_mesh,
    out_type=jax.ShapeDtypeStruct((NB, NF, 128), jnp.float32),
    compiler_params=pltpu.CompilerParams(
        needs_layout_passes=False, use_tc_tiling_on_sc=False),
    scratch_types=[
        pltpu.VMEM((CB, 3, 128), jnp.float32),
        *[pltpu.VMEM((C,), jnp.int32) for _ in range(16)],
        *[pltpu.VMEM((C,), jnp.float32) for _ in range(8)],
        *[pltpu.VMEM((C,), jnp.int32) for _ in range(8)],
        *[pltpu.VMEM((C, 8), jnp.float32) for _ in range(16)],
        pltpu.VMEM((CB, NF, 128), jnp.float32),
        pltpu.SemaphoreType.DMA,
    ],
)
def _ingp(x_hbm, table_hbm, out_hbm, xv, *rest):
    idxv = rest[0:16]
    wv = rest[16:24]
    lowv = rest[24:32]
    rows = rest[32:48]
    outv, sem = rest[48:]
    wid = lax.axis_index("s") * NC + lax.axis_index("c")
    iota = lax.iota(jnp.int32, 16)

    @pl.loop(np.int32(0), np.int32(NCHUNK),
             init_carry=wid * np.int32(NPW // 128))
    def chunk_body(ci, bblk):
        bblk = pl.multiple_of(bblk, CB)
        pltpu.sync_copy(x_hbm.at[pl.ds(bblk, CB)], xv)

        for b in range(CB):
            @pl.loop(np.int32(0), np.int32(8), init_carry=np.int32(0))
            def compute_body(i, wo):
                wo = pl.multiple_of(wo, 16)
                o = wo + np.int32(b * 128)
                t0 = xv[b, 0, pl.ds(wo, 16)] * RES
                t1 = xv[b, 1, pl.ds(wo, 16)] * RES
                t2 = xv[b, 2, pl.ds(wo, 16)] * RES
                c0 = t0.astype(jnp.int32)
                c1 = t1.astype(jnp.int32)
                c2 = t2.astype(jnp.int32)
                f0 = t0 - c0.astype(jnp.float32)
                f1 = t1 - c1.astype(jnp.float32)
                f2 = t2 - c2.astype(jnp.float32)
                u0 = c0.astype(jnp.uint32)
                u1 = c1.astype(jnp.uint32) * P2
                u2 = c2.astype(jnp.uint32) * P3
                u = ((u0, u0 + np.uint32(1)),
                     (u1, u1 + P2),
                     (u2, u2 + P3))
                g0 = 1.0 - f0
                g1 = 1.0 - f1
                g2 = 1.0 - f2
                wyz = {(0, 0): g1 * g2, (0, 1): g1 * f2,
                       (1, 0): f1 * g2, (1, 1): f1 * f2}
                wx = (g0, f0)
                for j, (a, bb, cc) in enumerate(CORNERS):
                    h = (u[0][a] ^ u[1][bb] ^ u[2][cc]) & HMASK
                    s3 = h >> np.uint32(3)
                    i0 = (((s3 >> np.uint32(4)) << np.uint32(5))
                          | (s3 & np.uint32(15))).astype(jnp.int32)
                    idxv[2 * j][pl.ds(o, 16)] = i0
                    idxv[2 * j + 1][pl.ds(o, 16)] = i0 + np.int32(16)
                    lowv[j][pl.ds(o, 16)] = (h & np.uint32(7)).astype(jnp.int32)
                    wv[j][pl.ds(o, 16)] = wx[a] * wyz[(bb, cc)]
                return wo + np.int32(16)

        descs = []
        for jf in range(16):
            descs.append(pltpu.async_copy(
                table_hbm.at[idxv[jf]], rows[jf], sem))
        for d in descs:
            d.wait()

        for b in range(CB):
            @pl.loop(np.int32(0), np.int32(8), init_carry=np.int32(0))
            def combine_body(i, wo):
                wo = pl.multiple_of(wo, 16)
                o = wo + np.int32(b * 128)
                lidx = o + iota
                acc0 = jnp.zeros((16,), jnp.float32)
                acc1 = jnp.zeros((16,), jnp.float32)
                for j in range(8):
                    wj = wv[j][pl.ds(o, 16)]
                    col = lowv[j][pl.ds(o, 16)]
                    acc0 = acc0 + wj * plsc.load_gather(rows[2 * j], [lidx, col])
                    acc1 = acc1 + wj * plsc.load_gather(rows[2 * j + 1], [lidx, col])
                outv[b, 0, pl.ds(wo, 16)] = acc0
                outv[b, 1, pl.ds(wo, 16)] = acc1
                return wo + np.int32(16)

        pltpu.sync_copy(outv, out_hbm.at[pl.ds(bblk, CB)])
        return bblk + np.int32(CB)


def kernel(x, table):
    from jax._src.config import enable_x64 as _x64_ctx
    with _x64_ctx(False):
        xb = x.reshape(NB, 128, 3).transpose(0, 2, 1)
        tp = table.reshape(TS // 128, 128, NF).transpose(0, 2, 1)
        tp = tp.reshape(TS // 4, 8)
        out3 = _ingp(xb, tp)
        return out3.transpose(0, 2, 1).reshape(BATCH, NF)


# speedup vs baseline: 193.3504x; 1.1047x over previous
import functools

import numpy as np
import jax
import jax.numpy as jnp
from jax import lax
from jax.experimental import pallas as pl
from jax.experimental.pallas import tpu as pltpu
from jax.experimental.pallas import tpu_sc as plsc

RES = 1024.0
TS = 4194304
HMASK = np.uint32(TS - 1)
P2 = np.uint32(2654435761)
P3 = np.uint32(805459861)
BATCH = 1048576
NF = 2
CORNERS = [(0, 0, 0), (0, 0, 1), (0, 1, 0), (0, 1, 1),
           (1, 0, 0), (1, 0, 1), (1, 1, 0), (1, 1, 1)]

NC, NS = 2, 16
NW = NC * NS
NPW = BATCH // NW
C = 512
CB = C // 128
NCHUNK = NPW // C
NB = BATCH // 128

_mesh = plsc.VectorSubcoreMesh(core_axis_name="c", subcore_axis_name="s")


@functools.partial(
    pl.kernel,
    mesh=_mesh,
    out_type=jax.ShapeDtypeStruct((NB, NF, 128), jnp.float32),
    compiler_params=pltpu.CompilerParams(
        needs_layout_passes=False, use_tc_tiling_on_sc=False),
    scratch_types=[
        *[pltpu.VMEM((CB, 3, 128), jnp.float32) for _ in range(2)],
        *[pltpu.VMEM((C,), jnp.int32) for _ in range(32)],
        *[pltpu.VMEM((C,), jnp.float32) for _ in range(16)],
        *[pltpu.VMEM((C,), jnp.int32) for _ in range(16)],
        *[pltpu.VMEM((C, 8), jnp.float32) for _ in range(16)],
        pltpu.VMEM((CB, NF, 128), jnp.float32),
        pltpu.SemaphoreType.DMA,
    ],
)
def _ingp(x_hbm, table_hbm, out_hbm, *rest):
    xv = rest[0:2]
    idxv = (rest[2:18], rest[18:34])
    wv = (rest[34:42], rest[42:50])
    lowv = (rest[50:58], rest[58:66])
    rows = rest[66:82]
    outv, sem = rest[82:]
    wid = lax.axis_index("s") * NC + lax.axis_index("c")
    iota = lax.iota(jnp.int32, 16)

    def loadx(p, bblk):
        pltpu.sync_copy(x_hbm.at[pl.ds(bblk, CB)], xv[p])

    def compute(p):
        for b in range(CB):
            @pl.loop(np.int32(0), np.int32(8), init_carry=np.int32(0))
            def compute_body(i, wo):
                wo = pl.multiple_of(wo, 16)
                o = wo + np.int32(b * 128)
                t0 = xv[p][b, 0, pl.ds(wo, 16)] * RES
                t1 = xv[p][b, 1, pl.ds(wo, 16)] * RES
                t2 = xv[p][b, 2, pl.ds(wo, 16)] * RES
                c0 = t0.astype(jnp.int32)
                c1 = t1.astype(jnp.int32)
                c2 = t2.astype(jnp.int32)
                f0 = t0 - c0.astype(jnp.float32)
                f1 = t1 - c1.astype(jnp.float32)
                f2 = t2 - c2.astype(jnp.float32)
                u0 = c0.astype(jnp.uint32)
                u1 = c1.astype(jnp.uint32) * P2
                u2 = c2.astype(jnp.uint32) * P3
                u = ((u0, u0 + np.uint32(1)),
                     (u1, u1 + P2),
                     (u2, u2 + P3))
                g0 = 1.0 - f0
                g1 = 1.0 - f1
                g2 = 1.0 - f2
                wyz = {(0, 0): g1 * g2, (0, 1): g1 * f2,
                       (1, 0): f1 * g2, (1, 1): f1 * f2}
                wx = (g0, f0)
                for j, (a, bb, cc) in enumerate(CORNERS):
                    h = (u[0][a] ^ u[1][bb] ^ u[2][cc]) & HMASK
                    s3 = h >> np.uint32(3)
                    i0 = (((s3 >> np.uint32(4)) << np.uint32(5))
                          | (s3 & np.uint32(15))).astype(jnp.int32)
                    idxv[p][2 * j][pl.ds(o, 16)] = i0
                    idxv[p][2 * j + 1][pl.ds(o, 16)] = i0 + np.int32(16)
                    lowv[p][j][pl.ds(o, 16)] = (
                        h & np.uint32(7)).astype(jnp.int32)
                    wv[p][j][pl.ds(o, 16)] = wx[a] * wyz[(bb, cc)]
                return wo + np.int32(16)

    def fire(p):
        return [pltpu.async_copy(table_hbm.at[idxv[p][jf]], rows[jf], sem)
                for jf in range(16)]

    def combine_out(p, bblk):
        for b in range(CB):
            @pl.loop(np.int32(0), np.int32(8), init_carry=np.int32(0))
            def combine_body(i, wo):
                wo = pl.multiple_of(wo, 16)
                o = wo + np.int32(b * 128)
                lidx = o + iota
                acc0 = jnp.zeros((16,), jnp.float32)
                acc1 = jnp.zeros((16,), jnp.float32)
                for j in range(8):
                    wj = wv[p][j][pl.ds(o, 16)]
                    col = lowv[p][j][pl.ds(o, 16)]
                    acc0 = acc0 + wj * plsc.load_gather(
                        rows[2 * j], [lidx, col])
                    acc1 = acc1 + wj * plsc.load_gather(
                        rows[2 * j + 1], [lidx, col])
                outv[b, 0, pl.ds(wo, 16)] = acc0
                outv[b, 1, pl.ds(wo, 16)] = acc1
                return wo + np.int32(16)

        pltpu.sync_copy(outv, out_hbm.at[pl.ds(bblk, CB)])

    base0 = wid * np.int32(NPW // 128)
    loadx(0, base0)
    compute(0)

    @pl.loop(np.int32(0), np.int32(NCHUNK // 2 - 1), init_carry=base0)
    def chunk_pair(ci, bblk):
        bblk = pl.multiple_of(bblk, CB)
        descs_a = fire(0)
        loadx(1, bblk + np.int32(CB))
        compute(1)
        for d in descs_a:
            d.wait()
        combine_out(0, bblk)
        descs_b = fire(1)
        loadx(0, bblk + np.int32(2 * CB))
        compute(0)
        for d in descs_b:
            d.wait()
        combine_out(1, bblk + np.int32(CB))
        return bblk + np.int32(2 * CB)

    last = pl.multiple_of(base0 + np.int32((NCHUNK - 2) * CB), CB)
    descs_a = fire(0)
    loadx(1, last + np.int32(CB))
    compute(1)
    for d in descs_a:
        d.wait()
    combine_out(0, last)
    descs_b = fire(1)
    for d in descs_b:
        d.wait()
    combine_out(1, last + np.int32(CB))


def kernel(x, table):
    from jax._src.config import enable_x64 as _x64_ctx
    with _x64_ctx(False):
        xb = x.reshape(NB, 128, 3).transpose(0, 2, 1)
        tp = table.reshape(TS // 128, 128, NF).transpose(0, 2, 1)
        tp = tp.reshape(TS // 4, 8)
        out3 = _ingp(xb, tp)
        return out3.transpose(0, 2, 1).reshape(BATCH, NF)


# speedup vs baseline: 220.3373x; 1.1396x over previous
import functools

import numpy as np
import jax
import jax.numpy as jnp
from jax import lax
from jax.experimental import pallas as pl
from jax.experimental.pallas import tpu as pltpu
from jax.experimental.pallas import tpu_sc as plsc

RES = 1024.0
TS = 4194304
HMASK = np.uint32(TS - 1)
P2 = np.uint32(2654435761)
P3 = np.uint32(805459861)
BATCH = 1048576
NF = 2
CORNERS = [(0, 0, 0), (0, 0, 1), (0, 1, 0), (0, 1, 1),
           (1, 0, 0), (1, 0, 1), (1, 1, 0), (1, 1, 1)]

NC, NS = 2, 16
NW = NC * NS
NPW = BATCH // NW
C = 256
CB = C // 128
NCHUNK = NPW // C
NB = BATCH // 128

_mesh = plsc.VectorSubcoreMesh(core_axis_name="c", subcore_axis_name="s")


@functools.partial(
    pl.kernel,
    mesh=_mesh,
    out_type=jax.ShapeDtypeStruct((NB, NF, 128), jnp.float32),
    compiler_params=pltpu.CompilerParams(
        needs_layout_passes=False, use_tc_tiling_on_sc=False),
    scratch_types=[
        *[pltpu.VMEM((CB, 3, 128), jnp.float32) for _ in range(2)],
        *[pltpu.VMEM((C,), jnp.int32) for _ in range(32)],
        *[pltpu.VMEM((C,), jnp.float32) for _ in range(16)],
        *[pltpu.VMEM((C,), jnp.int32) for _ in range(16)],
        *[pltpu.VMEM((C, 8), jnp.float32) for _ in range(32)],
        *[pltpu.VMEM((CB, NF, 128), jnp.float32) for _ in range(2)],
        pltpu.SemaphoreType.DMA,
    ],
)
def _ingp(x_hbm, table_hbm, out_hbm, *rest):
    xv = rest[0:2]
    idxv = (rest[2:18], rest[18:34])
    wv = (rest[34:42], rest[42:50])
    lowv = (rest[50:58], rest[58:66])
    rows = (rest[66:82], rest[82:98])
    outv = rest[98:100]
    sem = rest[100]
    wid = lax.axis_index("s") * NC + lax.axis_index("c")
    iota = lax.iota(jnp.int32, 16)

    def loadx(p, bblk):
        pltpu.sync_copy(x_hbm.at[pl.ds(bblk, CB)], xv[p])

    def compute(p):
        for b in range(CB):
            @pl.loop(np.int32(0), np.int32(8), init_carry=np.int32(0))
            def compute_body(i, wo):
                wo = pl.multiple_of(wo, 16)
                o = wo + np.int32(b * 128)
                t0 = xv[p][b, 0, pl.ds(wo, 16)] * RES
                t1 = xv[p][b, 1, pl.ds(wo, 16)] * RES
                t2 = xv[p][b, 2, pl.ds(wo, 16)] * RES
                c0 = t0.astype(jnp.int32)
                c1 = t1.astype(jnp.int32)
                c2 = t2.astype(jnp.int32)
                f0 = t0 - c0.astype(jnp.float32)
                f1 = t1 - c1.astype(jnp.float32)
                f2 = t2 - c2.astype(jnp.float32)
                u0 = c0.astype(jnp.uint32)
                u1 = c1.astype(jnp.uint32) * P2
                u2 = c2.astype(jnp.uint32) * P3
                u = ((u0, u0 + np.uint32(1)),
                     (u1, u1 + P2),
                     (u2, u2 + P3))
                g0 = 1.0 - f0
                g1 = 1.0 - f1
                g2 = 1.0 - f2
                wyz = {(0, 0): g1 * g2, (0, 1): g1 * f2,
                       (1, 0): f1 * g2, (1, 1): f1 * f2}
                wx = (g0, f0)
                for j, (a, bb, cc) in enumerate(CORNERS):
                    h = (u[0][a] ^ u[1][bb] ^ u[2][cc]) & HMASK
                    s3 = h >> np.uint32(3)
                    i0 = (((s3 >> np.uint32(4)) << np.uint32(5))
                          | (s3 & np.uint32(15))).astype(jnp.int32)
                    idxv[p][2 * j][pl.ds(o, 16)] = i0
                    idxv[p][2 * j + 1][pl.ds(o, 16)] = i0 + np.int32(16)
                    lowv[p][j][pl.ds(o, 16)] = (
                        h & np.uint32(7)).astype(jnp.int32)
                    wv[p][j][pl.ds(o, 16)] = wx[a] * wyz[(bb, cc)]
                return wo + np.int32(16)

    def fire(p):
        for jf in range(16):
            pltpu.async_copy(table_hbm.at[idxv[p][jf]], rows[p][jf], sem)

    def wait_gathers(p):
        for jf in range(16):
            pltpu.make_async_copy(
                table_hbm.at[idxv[p][jf]], rows[p][jf], sem).wait()

    def combine_out(p, bblk):
        for b in range(CB):
            @pl.loop(np.int32(0), np.int32(8), init_carry=np.int32(0))
            def combine_body(i, wo):
                wo = pl.multiple_of(wo, 16)
                o = wo + np.int32(b * 128)
                lidx = o + iota
                acc0 = jnp.zeros((16,), jnp.float32)
                acc1 = jnp.zeros((16,), jnp.float32)
                for j in range(8):
                    wj = wv[p][j][pl.ds(o, 16)]
                    col = lowv[p][j][pl.ds(o, 16)]
                    acc0 = acc0 + wj * plsc.load_gather(
                        rows[p][2 * j], [lidx, col])
                    acc1 = acc1 + wj * plsc.load_gather(
                        rows[p][2 * j + 1], [lidx, col])
                outv[p][b, 0, pl.ds(wo, 16)] = acc0
                outv[p][b, 1, pl.ds(wo, 16)] = acc1
                return wo + np.int32(16)

        pltpu.sync_copy(outv[p], out_hbm.at[pl.ds(bblk, CB)])

    base0 = wid * np.int32(NPW // 128)
    loadx(0, base0)
    compute(0)
    fire(0)

    @pl.loop(np.int32(0), np.int32(NCHUNK // 2 - 1), init_carry=base0)
    def chunk_pair(ci, bblk):
        bblk = pl.multiple_of(bblk, CB)
        loadx(1, bblk + np.int32(CB))
        compute(1)
        fire(1)
        wait_gathers(0)
        combine_out(0, bblk)
        loadx(0, bblk + np.int32(2 * CB))
        compute(0)
        fire(0)
        wait_gathers(1)
        combine_out(1, bblk + np.int32(CB))
        return bblk + np.int32(2 * CB)

    last = pl.multiple_of(base0 + np.int32((NCHUNK - 2) * CB), CB)
    loadx(1, last + np.int32(CB))
    compute(1)
    fire(1)
    wait_gathers(0)
    combine_out(0, last)
    wait_gathers(1)
    combine_out(1, last + np.int32(CB))


def kernel(x, table):
    from jax._src.config import enable_x64 as _x64_ctx
    with _x64_ctx(False):
        xb = x.reshape(NB, 128, 3).transpose(0, 2, 1)
        tp = table.reshape(TS // 128, 128, NF).transpose(0, 2, 1)
        tp = tp.reshape(TS // 4, 8)
        out3 = _ingp(xb, tp)
        return out3.transpose(0, 2, 1).reshape(BATCH, NF)


# speedup vs baseline: 333.4590x; 1.5134x over previous
import functools

import numpy as np
import jax
import jax.numpy as jnp
from jax import lax
from jax.experimental import pallas as pl
from jax.experimental.pallas import tpu as pltpu
from jax.experimental.pallas import tpu_sc as plsc

RES = 1024.0
TS = 4194304
HMASK = np.uint32(TS - 1)
P2 = np.uint32(2654435761)
P3 = np.uint32(805459861)
BATCH = 1048576
NF = 2
CORNERS = [(0, 0, 0), (0, 0, 1), (0, 1, 0), (0, 1, 1),
           (1, 0, 0), (1, 0, 1), (1, 1, 0), (1, 1, 1)]

NC, NS = 2, 16
NW = NC * NS
NPW = BATCH // NW
C = 512
CB = C // 128
NCHUNK = NPW // C
NB = BATCH // 128

_mesh = plsc.VectorSubcoreMesh(core_axis_name="c", subcore_axis_name="s")

NBLK = TS // 128
RB = 128
RBAT = NBLK // NW // RB


@functools.partial(
    pl.kernel,
    mesh=_mesh,
    out_type=jax.ShapeDtypeStruct((TS // 4, 8), jnp.float32),
    compiler_params=pltpu.CompilerParams(
        needs_layout_passes=False, use_tc_tiling_on_sc=False),
    scratch_types=[
        pltpu.VMEM((RB, 2, 128), jnp.float32),
        pltpu.VMEM((RB * 256 // 8, 8), jnp.float32),
        pltpu.SemaphoreType.DMA,
    ],
)
def _repack(tp_hbm, t2_hbm, srcv, dstv, sem):
    wid = lax.axis_index("s") * NC + lax.axis_index("c")
    iota = lax.iota(jnp.int32, 16)
    iota2 = iota * np.int32(2)

    @pl.loop(np.int32(0), np.int32(RBAT),
             init_carry=wid * np.int32(RBAT * RB))
    def batch_body(bi, blk0):
        blk0 = pl.multiple_of(blk0, RB)
        pltpu.sync_copy(tp_hbm.at[pl.ds(blk0, RB)], srcv)

        for b in range(RB):
            @pl.loop(np.int32(0), np.int32(8), init_carry=np.int32(0))
            def il_body(i, wo):
                wo = pl.multiple_of(wo, 16)
                f0 = srcv[b, 0, pl.ds(wo, 16)]
                f1 = srcv[b, 1, pl.ds(wo, 16)]
                pos0 = wo * np.int32(2) + np.int32(b * 256) + iota2
                r0 = pos0 >> np.int32(3)
                c0 = pos0 & np.int32(7)
                plsc.store_scatter(dstv, [r0, c0], f0)
                pos1 = pos0 + np.int32(1)
                plsc.store_scatter(dstv, [pos1 >> np.int32(3),
                                          pos1 & np.int32(7)], f1)
                return wo + np.int32(16)

        pltpu.sync_copy(dstv, t2_hbm.at[pl.ds(blk0 * np.int32(32),
                                              RB * 32)])
        return blk0 + np.int32(RB)


@functools.partial(
    pl.kernel,
    mesh=_mesh,
    out_type=jax.ShapeDtypeStruct((NB, NF, 128), jnp.float32),
    compiler_params=pltpu.CompilerParams(
        needs_layout_passes=False, use_tc_tiling_on_sc=False),
    scratch_types=[
        *[pltpu.VMEM((CB, 3, 128), jnp.float32) for _ in range(2)],
        *[pltpu.VMEM((C,), jnp.int32) for _ in range(16)],
        *[pltpu.VMEM((C,), jnp.float32) for _ in range(16)],
        *[pltpu.VMEM((C,), jnp.int32) for _ in range(16)],
        *[pltpu.VMEM((C, 8), jnp.float32) for _ in range(16)],
        *[pltpu.VMEM((CB, NF, 128), jnp.float32) for _ in range(2)],
        pltpu.SemaphoreType.DMA,
    ],
)
def _ingp(x_hbm, table_hbm, out_hbm, *rest):
    xv = rest[0:2]
    idxv = (rest[2:10], rest[10:18])
    wv = (rest[18:26], rest[26:34])
    lowv = (rest[34:42], rest[42:50])
    rows = (rest[50:58], rest[58:66])
    outv = rest[66:68]
    sem = rest[68]
    wid = lax.axis_index("s") * NC + lax.axis_index("c")
    iota = lax.iota(jnp.int32, 16)

    def loadx(p, bblk):
        pltpu.sync_copy(x_hbm.at[pl.ds(bblk, CB)], xv[p])

    def compute(p):
        for b in range(CB):
            @pl.loop(np.int32(0), np.int32(8), init_carry=np.int32(0))
            def compute_body(i, wo):
                wo = pl.multiple_of(wo, 16)
                o = wo + np.int32(b * 128)
                t0 = xv[p][b, 0, pl.ds(wo, 16)] * RES
                t1 = xv[p][b, 1, pl.ds(wo, 16)] * RES
                t2 = xv[p][b, 2, pl.ds(wo, 16)] * RES
                c0 = t0.astype(jnp.int32)
                c1 = t1.astype(jnp.int32)
                c2 = t2.astype(jnp.int32)
                f0 = t0 - c0.astype(jnp.float32)
                f1 = t1 - c1.astype(jnp.float32)
                f2 = t2 - c2.astype(jnp.float32)
                u0 = c0.astype(jnp.uint32)
                u1 = c1.astype(jnp.uint32) * P2
                u2 = c2.astype(jnp.uint32) * P3
                u = ((u0, u0 + np.uint32(1)),
                     (u1, u1 + P2),
                     (u2, u2 + P3))
                g0 = 1.0 - f0
                g1 = 1.0 - f1
                g2 = 1.0 - f2
                wyz = {(0, 0): g1 * g2, (0, 1): g1 * f2,
                       (1, 0): f1 * g2, (1, 1): f1 * f2}
                wx = (g0, f0)
                for j, (a, bb, cc) in enumerate(CORNERS):
                    h = (u[0][a] ^ u[1][bb] ^ u[2][cc]) & HMASK
                    i0 = (h >> np.uint32(2)).astype(jnp.int32)
                    idxv[p][j][pl.ds(o, 16)] = i0
                    lowv[p][j][pl.ds(o, 16)] = (
                        (h & np.uint32(3)) * np.uint32(2)).astype(jnp.int32)
                    wv[p][j][pl.ds(o, 16)] = wx[a] * wyz[(bb, cc)]
                return wo + np.int32(16)

    def fire(p):
        for jf in range(8):
            pltpu.async_copy(table_hbm.at[idxv[p][jf]], rows[p][jf], sem)

    def wait_gathers(p):
        for jf in range(8):
            pltpu.make_async_copy(
                table_hbm.at[idxv[p][jf]], rows[p][jf], sem).wait()

    def combine_out(p, bblk):
        for b in range(CB):
            @pl.loop(np.int32(0), np.int32(8), init_carry=np.int32(0))
            def combine_body(i, wo):
                wo = pl.multiple_of(wo, 16)
                o = wo + np.int32(b * 128)
                lidx = o + iota
                acc0 = jnp.zeros((16,), jnp.float32)
                acc1 = jnp.zeros((16,), jnp.float32)
                one16 = jnp.ones((16,), jnp.int32)
                for j in range(8):
                    wj = wv[p][j][pl.ds(o, 16)]
                    col = lowv[p][j][pl.ds(o, 16)]
                    acc0 = acc0 + wj * plsc.load_gather(
                        rows[p][j], [lidx, col])
                    acc1 = acc1 + wj * plsc.load_gather(
                        rows[p][j], [lidx, col + one16])
                outv[p][b, 0, pl.ds(wo, 16)] = acc0
                outv[p][b, 1, pl.ds(wo, 16)] = acc1
                return wo + np.int32(16)

        pltpu.sync_copy(outv[p], out_hbm.at[pl.ds(bblk, CB)])

    base0 = wid * np.int32(NPW // 128)
    loadx(0, base0)
    compute(0)
    fire(0)

    @pl.loop(np.int32(0), np.int32(NCHUNK // 2 - 1), init_carry=base0)
    def chunk_pair(ci, bblk):
        bblk = pl.multiple_of(bblk, CB)
        loadx(1, bblk + np.int32(CB))
        compute(1)
        fire(1)
        wait_gathers(0)
        combine_out(0, bblk)
        loadx(0, bblk + np.int32(2 * CB))
        compute(0)
        fire(0)
        wait_gathers(1)
        combine_out(1, bblk + np.int32(CB))
        return bblk + np.int32(2 * CB)

    last = pl.multiple_of(base0 + np.int32((NCHUNK - 2) * CB), CB)
    loadx(1, last + np.int32(CB))
    compute(1)
    fire(1)
    wait_gathers(0)
    combine_out(0, last)
    wait_gathers(1)
    combine_out(1, last + np.int32(CB))


def kernel(x, table):
    from jax._src.config import enable_x64 as _x64_ctx
    with _x64_ctx(False):
        xb = x.reshape(NB, 128, 3).transpose(0, 2, 1)
        tp = table.reshape(TS // 128, 128, NF).transpose(0, 2, 1)
        t2 = _repack(tp)
        out3 = _ingp(xb, t2)
        return out3.transpose(0, 2, 1).reshape(BATCH, NF)
